# Initial kernel scaffold; baseline (speedup 1.0000x reference)
import jax
import jax.numpy as jnp
from jax.experimental import pallas as pl


def kernel(x, edge_index, edge_weight, W1, b1, W2, b2, W3, b3, Wo, bo):
    raise NotImplementedError("write your pallas kernel here")


# speedup vs baseline: 6.8522x; 6.8522x over previous
import functools

import jax
import jax.numpy as jnp
from jax import lax
from jax.experimental import pallas as pl
from jax.experimental.pallas import tpu as pltpu
from jax.experimental.pallas import tpu_sc as plsc

N = 10000
E = 160000
D = 256
P = 12
H = 128
NSUB = 16
EPT = E // NSUB
ROFF = 624
RCNT = 640
CHUNK = 80
NCHUNK = EPT // CHUNK

_mesh = plsc.VectorSubcoreMesh(core_axis_name="c", subcore_axis_name="s")


@functools.partial(
    pl.kernel,
    out_type=(jax.ShapeDtypeStruct((N, H), jnp.float32),
              jax.ShapeDtypeStruct((N, H), jnp.float32)),
    mesh=_mesh,
    scratch_types=[
        pltpu.VMEM((EPT,), jnp.int32),
        pltpu.VMEM((EPT,), jnp.int32),
        pltpu.VMEM((EPT,), jnp.float32),
        pltpu.VMEM((CHUNK,), jnp.int32),
        pltpu.VMEM((CHUNK,), jnp.int32),
        pltpu.VMEM((CHUNK, H), jnp.float32),
        pltpu.VMEM_SHARED((N, H), jnp.float32),
        pltpu.SemaphoreType.DMA,
    ],
)
def _agg_kernel(p0_hbm, p1_hbm, src_hbm, dst_hbm, ew_hbm, out0_hbm, out1_hbm,
                src_v, dst_v, ew_v, sidx_v, didx_v, rows_v, agg_sh, sem):
    cid = lax.axis_index("c")
    sid = lax.axis_index("s")
    ebase = sid * EPT
    rbase = sid * ROFF

    def run(p_hbm, out_hbm):
        pltpu.sync_copy(src_hbm.at[pl.ds(ebase, EPT)], src_v)
        pltpu.sync_copy(dst_hbm.at[pl.ds(ebase, EPT)], dst_v)
        pltpu.sync_copy(ew_hbm.at[pl.ds(ebase, EPT)], ew_v)
        pltpu.sync_copy(p_hbm.at[pl.ds(rbase, RCNT)],
                        agg_sh.at[pl.ds(rbase, RCNT)])
        plsc.subcore_barrier()

        def chunk(ci, carry):
            cb = ci * CHUNK
            for k16 in range(CHUNK // 16):
                sidx_v[pl.ds(k16 * 16, 16)] = src_v[pl.ds(cb + k16 * 16, 16)]
                didx_v[pl.ds(k16 * 16, 16)] = dst_v[pl.ds(cb + k16 * 16, 16)]
            pltpu.async_copy(p_hbm.at[sidx_v], rows_v, sem).wait()
            for k16 in range(CHUNK // 16):
                w16 = ew_v[pl.ds(cb + k16 * 16, 16)]
                for j in range(16):
                    wj = w16[j]
                    r = k16 * 16 + j
                    for c8 in range(H // 16):
                        sl = pl.ds(c8 * 16, 16)
                        rows_v[r, sl] = rows_v[r, sl] * wj
            pltpu.sync_copy(rows_v, agg_sh.at[didx_v], add=True)
            return carry

        lax.fori_loop(0, NCHUNK, chunk, 0)
        plsc.subcore_barrier()
        pltpu.sync_copy(agg_sh.at[pl.ds(rbase, RCNT)],
                        out_hbm.at[pl.ds(rbase, RCNT)])

    @pl.when(cid == 0)
    def _():
        run(p0_hbm, out0_hbm)

    @pl.when(cid == 1)
    def _():
        run(p1_hbm, out1_hbm)


BN = 1000


def _dinv_of(deg_blk):
    deg = deg_blk[:, 0:1]
    return jnp.where(deg > 0, lax.rsqrt(deg), 0.0)


def _mm1_body(x_ref, dg_ref, w_ref, p0_ref, p1_ref):
    dinv = _dinv_of(dg_ref[...])
    m = jnp.dot(x_ref[...], w_ref[...], preferred_element_type=jnp.float32)
    p = m * dinv
    p0_ref[...] = p[:, :H]
    p1_ref[...] = p[:, H:]


def _layer_body(a0_ref, a1_ref, dg_ref, b_ref, w_ref, p0_ref, p1_ref):
    dinv = _dinv_of(dg_ref[...])
    h = jnp.concatenate([a0_ref[...], a1_ref[...]], axis=1)
    h = jnp.maximum(h * dinv + b_ref[...], 0.0)
    m = jnp.dot(h, w_ref[...], preferred_element_type=jnp.float32)
    p = m * dinv
    p0_ref[...] = p[:, :H]
    p1_ref[...] = p[:, H:]


def _head_body(a0_ref, a1_ref, dg_ref, b_ref, wo_ref, bo_ref, o_ref):
    dinv = _dinv_of(dg_ref[...])
    h = jnp.concatenate([a0_ref[...], a1_ref[...]], axis=1)
    h = jnp.maximum(h * dinv + b_ref[...], 0.0)
    o_ref[...] = jnp.dot(h, wo_ref[...],
                         preferred_element_type=jnp.float32) + bo_ref[...]


def _row_spec(w):
    return pl.BlockSpec((BN, w), lambda i: (i, 0))


def _full_spec(r, w):
    return pl.BlockSpec((r, w), lambda i: (0, 0))


_p_out = ([pl.BlockSpec((BN, H), lambda i: (i, 0)),
           pl.BlockSpec((BN, H), lambda i: (i, 0))],
          [jax.ShapeDtypeStruct((N, H), jnp.float32),
           jax.ShapeDtypeStruct((N, H), jnp.float32)])


def _mm1(x, deg16, W):
    out_specs, out_shape = _p_out
    return pl.pallas_call(
        _mm1_body,
        grid=(N // BN,),
        in_specs=[_row_spec(D), _row_spec(H), _full_spec(D, D)],
        out_specs=out_specs,
        out_shape=out_shape,
    )(x, deg16, W)


def _layer(a0, a1, deg16, b, W):
    out_specs, out_shape = _p_out
    return pl.pallas_call(
        _layer_body,
        grid=(N // BN,),
        in_specs=[_row_spec(H), _row_spec(H), _row_spec(H),
                  _full_spec(1, D), _full_spec(D, D)],
        out_specs=out_specs,
        out_shape=out_shape,
    )(a0, a1, deg16, b.reshape(1, D), W)


def _head(a0, a1, deg16, b, Wo, bo):
    return pl.pallas_call(
        _head_body,
        grid=(N // BN,),
        in_specs=[_row_spec(H), _row_spec(H), _row_spec(H),
                  _full_spec(1, D), _full_spec(D, P), _full_spec(1, P)],
        out_specs=pl.BlockSpec((BN, P), lambda i: (i, 0)),
        out_shape=jax.ShapeDtypeStruct((N, P), jnp.float32),
    )(a0, a1, deg16, b.reshape(1, D), Wo, bo.reshape(1, P))


def kernel(x, edge_index, edge_weight, W1, b1, W2, b2, W3, b3, Wo, bo):
    src = edge_index[0]
    dst = edge_index[1]

    ones = jnp.ones((N, H), jnp.float32)
    deg16, _ = _agg_kernel(ones, ones, src, dst, edge_weight)
    p0, p1 = _mm1(x, deg16, W1)
    a0, a1 = _agg_kernel(p0, p1, src, dst, edge_weight)
    p0, p1 = _layer(a0, a1, deg16, b1, W2)
    a0, a1 = _agg_kernel(p0, p1, src, dst, edge_weight)
    p0, p1 = _layer(a0, a1, deg16, b2, W3)
    a0, a1 = _agg_kernel(p0, p1, src, dst, edge_weight)
    return _head(a0, a1, deg16, b3, Wo, bo)


# speedup vs baseline: 12.5793x; 1.8358x over previous
import functools

import jax
import jax.numpy as jnp
from jax import lax
from jax.experimental import pallas as pl
from jax.experimental.pallas import tpu as pltpu
from jax.experimental.pallas import tpu_sc as plsc

N = 10000
E = 160000
D = 256
P = 12
H = 128
NSUB = 16
EPT = E // NSUB
ROFF = 624
RCNT = 640
CHUNK = 80
NCHUNK = EPT // CHUNK
NPAIR = (NCHUNK - 1) // 2

_mesh = plsc.VectorSubcoreMesh(core_axis_name="c", subcore_axis_name="s")

E_DEG = 160256
EPT_DEG = E_DEG // 32
NW = 32


@functools.partial(
    pl.kernel,
    out_type=jax.ShapeDtypeStruct((NW, N), jnp.float32),
    mesh=_mesh,
    compiler_params=pltpu.CompilerParams(needs_layout_passes=False),
    scratch_types=[
        pltpu.VMEM((EPT_DEG,), jnp.int32),
        pltpu.VMEM((EPT_DEG,), jnp.float32),
        pltpu.VMEM((N,), jnp.float32),
    ],
)
def _deg_kernel(dst_hbm, ew_hbm, out_hbm, dst_v, ew_v, hist_v):
    cid = lax.axis_index("c")
    sid = lax.axis_index("s")
    wid = sid * 2 + cid
    ebase = wid * EPT_DEG

    pltpu.sync_copy(dst_hbm.at[pl.ds(ebase, EPT_DEG)], dst_v)
    pltpu.sync_copy(ew_hbm.at[pl.ds(ebase, EPT_DEG)], ew_v)

    def zero(i, carry):
        hist_v[pl.ds(i * 16, 16)] = jnp.zeros((16,), jnp.float32)
        return carry

    lax.fori_loop(0, N // 16, zero, 0)

    def accum(k, carry):
        idx16 = dst_v[pl.ds(k * 16, 16)]
        w16 = ew_v[pl.ds(k * 16, 16)]
        plsc.addupdate_scatter(hist_v, [idx16], w16)
        return carry

    lax.fori_loop(0, EPT_DEG // 16, accum, 0)
    pltpu.sync_copy(hist_v, out_hbm.at[wid])


@functools.partial(
    pl.kernel,
    out_type=jax.ShapeDtypeStruct((2 * N, H), jnp.float32),
    mesh=_mesh,
    scratch_types=[
        pltpu.VMEM((NCHUNK, CHUNK), jnp.int32),
        pltpu.VMEM((3, CHUNK), jnp.int32),
        pltpu.VMEM((3, CHUNK), jnp.float32),
        pltpu.VMEM((CHUNK, H), jnp.float32),
        pltpu.VMEM((CHUNK, H), jnp.float32),
        pltpu.VMEM((CHUNK, H), jnp.float32),
        pltpu.VMEM_SHARED((N, H), jnp.float32),
        pltpu.SemaphoreType.DMA,
        pltpu.SemaphoreType.DMA,
        pltpu.SemaphoreType.DMA,
        pltpu.SemaphoreType.DMA,
        pltpu.SemaphoreType.DMA,
        pltpu.SemaphoreType.DMA,
        pltpu.SemaphoreType.DMA,
        pltpu.SemaphoreType.DMA,
        pltpu.SemaphoreType.DMA,
        pltpu.SemaphoreType.DMA,
        pltpu.SemaphoreType.DMA,
        pltpu.SemaphoreType.DMA,
    ],
)
def _agg_kernel(p_hbm, src_hbm, dst3_hbm, ew_hbm, out_hbm,
                dsts, gi3, ew3, rows0, rows1, rows2, agg_sh,
                gs0, gs1, gs2, ss0, ss1, ss2,
                ps0, ps1, ps2, pe0, pe1, pe2):
    cid = lax.axis_index("c")
    sid = lax.axis_index("s")
    ebase = sid * EPT
    rbase = sid * ROFF
    pbase = cid * N

    rows = (rows0, rows1, rows2)
    gsem = (gs0, gs1, gs2)
    ssem = (ss0, ss1, ss2)
    psem = (ps0, ps1, ps2)
    pesem = (pe0, pe1, pe2)

    pltpu.sync_copy(dst3_hbm.at[sid], dsts)
    pltpu.sync_copy(p_hbm.at[pl.ds(pbase + rbase, RCNT)],
                    agg_sh.at[pl.ds(rbase, RCNT)])
    plsc.subcore_barrier()

    def prefetch(ci, slot):
        cb = ci * CHUNK
        pltpu.async_copy(src_hbm.at[pl.ds(ebase + cb, CHUNK)],
                         gi3.at[slot], psem[slot])
        pltpu.async_copy(ew_hbm.at[pl.ds(ebase + cb, CHUNK)],
                         ew3.at[slot], pesem[slot])

    def start_gather(b):
        pltpu.make_async_copy(src_hbm.at[pl.ds(0, CHUNK)],
                              gi3.at[b], psem[b]).wait()
        gi = gi3.at[b]
        for k16 in range(CHUNK // 16):
            sl = pl.ds(k16 * 16, 16)
            gi[sl] = gi[sl] + pbase
        pltpu.async_copy(p_hbm.at[gi], rows[b], gsem[b])

    def scale(b):
        pltpu.make_async_copy(ew_hbm.at[pl.ds(0, CHUNK)],
                              ew3.at[b], pesem[b]).wait()
        for k16 in range(CHUNK // 16):
            w16 = ew3[b, pl.ds(k16 * 16, 16)]
            ws = [w16[j] for j in range(16)]
            for c8 in range(H // 16):
                sl = pl.ds(c8 * 16, 16)
                for j in range(16):
                    r = k16 * 16 + j
                    rows[b][r, sl] = rows[b][r, sl] * ws[j]

    def process(ci, b, first_round, prefetch_next=True):
        nb = (b + 1) % 3
        nnb = (b + 2) % 3
        if not first_round:
            pltpu.make_async_copy(rows[nb], agg_sh.at[dsts.at[ci - 2]],
                                  ssem[nb]).wait()
        start_gather(nb)
        if prefetch_next:
            prefetch(ci + 2, nnb)
        pltpu.make_async_copy(p_hbm.at[gi3.at[b]], rows[b], gsem[b]).wait()
        scale(b)
        pltpu.async_copy(rows[b], agg_sh.at[dsts.at[ci]], ssem[b], add=True)

    prefetch(0, 0)
    prefetch(1, 1)
    start_gather(0)
    process(0, 0, True)
    process(1, 1, True)

    def triple(j, carry):
        c = 3 * j + 2
        process(c, 2, False)
        process(c + 1, 0, False)
        process(c + 2, 1, False)
        return carry

    lax.fori_loop(0, (NCHUNK - 5) // 3, triple, 0)

    process(NCHUNK - 3, 2, False)
    process(NCHUNK - 2, 0, False, prefetch_next=False)
    pltpu.make_async_copy(rows[2], agg_sh.at[dsts.at[NCHUNK - 3]],
                          ssem[2]).wait()
    pltpu.make_async_copy(p_hbm.at[gi3.at[1]], rows[1], gsem[1]).wait()
    scale(1)
    pltpu.async_copy(rows[1], agg_sh.at[dsts.at[NCHUNK - 1]], ssem[1],
                     add=True)
    pltpu.make_async_copy(rows[0], agg_sh.at[dsts.at[NCHUNK - 2]],
                          ssem[0]).wait()
    pltpu.make_async_copy(rows[1], agg_sh.at[dsts.at[NCHUNK - 1]],
                          ssem[1]).wait()
    plsc.subcore_barrier()
    pltpu.sync_copy(agg_sh.at[pl.ds(rbase, RCNT)],
                    out_hbm.at[pl.ds(pbase + rbase, RCNT)])


BN = 1000
NB = N // BN


def _deg_reduce_body(parts_ref, deg_ref):
    s = jnp.sum(parts_ref[...], axis=0) + 1.0
    deg_ref[...] = jnp.broadcast_to(s[:, None], (N, 8))


def _deg_reduce(parts):
    return pl.pallas_call(
        _deg_reduce_body,
        out_shape=jax.ShapeDtypeStruct((N, 8), jnp.float32),
    )(parts)


def _dinv_of(deg_blk):
    deg = deg_blk[:, 0:1]
    return jnp.where(deg > 0, lax.rsqrt(deg), 0.0)


def _mm1_body(x_ref, dg_ref, w_ref, p_ref):
    dinv = _dinv_of(dg_ref[...])
    m = jnp.dot(x_ref[...], w_ref[...], preferred_element_type=jnp.float32)
    p = m * dinv
    p_ref[0] = p[:, :H]
    p_ref[1] = p[:, H:]


def _layer_body(a0_ref, a1_ref, dg_ref, b_ref, w_ref, p_ref):
    dinv = _dinv_of(dg_ref[...])
    h = jnp.concatenate([a0_ref[...], a1_ref[...]], axis=1)
    h = jnp.maximum(h * dinv + b_ref[...], 0.0)
    m = jnp.dot(h, w_ref[...], preferred_element_type=jnp.float32)
    p = m * dinv
    p_ref[0] = p[:, :H]
    p_ref[1] = p[:, H:]


def _head_body(a0_ref, a1_ref, dg_ref, b_ref, wo_ref, bo_ref, o_ref):
    dinv = _dinv_of(dg_ref[...])
    h = jnp.concatenate([a0_ref[...], a1_ref[...]], axis=1)
    h = jnp.maximum(h * dinv + b_ref[...], 0.0)
    o_ref[...] = jnp.dot(h, wo_ref[...],
                         preferred_element_type=jnp.float32) + bo_ref[...]


def _half0_spec(w=H):
    return pl.BlockSpec((BN, w), lambda i: (i, 0))


def _deg_spec():
    return pl.BlockSpec((BN, 8), lambda i: (i, 0))


def _half1_spec():
    return pl.BlockSpec((BN, H), lambda i: (i + NB, 0))


def _full_spec(r, w):
    return pl.BlockSpec((r, w), lambda i: (0, 0))


_p_out_spec = pl.BlockSpec((2, BN, H), lambda i: (0, i, 0))
_p_out_shape = jax.ShapeDtypeStruct((2, N, H), jnp.float32)


def _mm1(x, deg, W):
    p = pl.pallas_call(
        _mm1_body,
        grid=(NB,),
        in_specs=[_half0_spec(D), _deg_spec(), _full_spec(D, D)],
        out_specs=_p_out_spec,
        out_shape=_p_out_shape,
    )(x, deg, W)
    return p.reshape(2 * N, H)


def _layer(agg, deg, b, W):
    p = pl.pallas_call(
        _layer_body,
        grid=(NB,),
        in_specs=[_half0_spec(), _half1_spec(), _deg_spec(),
                  _full_spec(1, D), _full_spec(D, D)],
        out_specs=_p_out_spec,
        out_shape=_p_out_shape,
    )(agg, agg, deg, b.reshape(1, D), W)
    return p.reshape(2 * N, H)


def _head(agg, deg, b, Wo, bo):
    return pl.pallas_call(
        _head_body,
        grid=(NB,),
        in_specs=[_half0_spec(), _half1_spec(), _deg_spec(),
                  _full_spec(1, D), _full_spec(D, P), _full_spec(1, P)],
        out_specs=pl.BlockSpec((BN, P), lambda i: (i, 0)),
        out_shape=jax.ShapeDtypeStruct((N, P), jnp.float32),
    )(agg, agg, deg, b.reshape(1, D), Wo, bo.reshape(1, P))


def kernel(x, edge_index, edge_weight, W1, b1, W2, b2, W3, b3, Wo, bo):
    src = edge_index[0]
    dst = edge_index[1]

    pad = E_DEG - E
    dstp = jnp.concatenate([dst, jnp.zeros((pad,), jnp.int32)])
    ewp = jnp.concatenate([edge_weight, jnp.zeros((pad,), jnp.float32)])
    deg = _deg_reduce(_deg_kernel(dstp, ewp))
    dst3 = dst.reshape(NSUB, NCHUNK, CHUNK)
    p = _mm1(x, deg, W1)
    agg = _agg_kernel(p, src, dst3, edge_weight)
    p = _layer(agg, deg, b1, W2)
    agg = _agg_kernel(p, src, dst3, edge_weight)
    p = _layer(agg, deg, b2, W3)
    agg = _agg_kernel(p, src, dst3, edge_weight)
    return _head(agg, deg, b3, Wo, bo)
